# Initial kernel scaffold; baseline (speedup 1.0000x reference)
import jax
import jax.numpy as jnp
from jax.experimental import pallas as pl


def kernel(x, edge_index, mask, batch, W0, b0, g0, be0, W1, b1, g1, be1, W2, b2, g2, be2, Wout, bout):
    raise NotImplementedError("write your pallas kernel here")


# speedup vs baseline: 10.1080x; 10.1080x over previous
import functools

import jax
import jax.numpy as jnp
from jax import lax
from jax.experimental import pallas as pl
from jax.experimental.pallas import tpu as pltpu
from jax.experimental.pallas import tpu_sc as plsc

N = 10000
F = 128
G = 16
NUM_CORES = 2
NUM_SUBCORES = 16
NUM_TILES = NUM_CORES * NUM_SUBCORES
NPAD = 10240
RPT = NPAD // NUM_SUBCORES
CHUNK = 80
DEG_W = 16


def _sc_mesh():
    return plsc.VectorSubcoreMesh(core_axis_name="c", subcore_axis_name="s")


def _make_deg_kernel(E):
    ept = E // NUM_TILES
    assert ept % CHUNK == 0
    n_chunks = ept // CHUNK

    @functools.partial(
        pl.kernel,
        out_type=jax.ShapeDtypeStruct((NUM_CORES * NPAD, DEG_W), jnp.float32),
        mesh=_sc_mesh(),
        scratch_types=[
            pltpu.VMEM((CHUNK,), jnp.int32),
            pltpu.VMEM((CHUNK, DEG_W), jnp.float32),
            pltpu.VMEM_SHARED((NPAD, DEG_W), jnp.float32),
        ],
    )
    def deg_kernel(dst_hbm, ones_hbm, zeros_hbm, out_hbm, didx, ones_v, acc):
        c = lax.axis_index("c")
        s = lax.axis_index("s")
        tile = c * NUM_SUBCORES + s
        pltpu.sync_copy(ones_hbm, ones_v)
        pltpu.sync_copy(
            zeros_hbm.at[pl.ds(s * RPT, RPT)], acc.at[pl.ds(s * RPT, RPT)]
        )
        plsc.subcore_barrier()
        base = tile * ept

        def body(i, carry):
            off = base + i * CHUNK
            pltpu.sync_copy(dst_hbm.at[pl.ds(off, CHUNK)], didx)
            pltpu.sync_copy(ones_v, acc.at[didx], add=True)
            return carry

        lax.fori_loop(0, n_chunks, body, 0)
        plsc.subcore_barrier()
        pltpu.sync_copy(
            acc.at[pl.ds(s * RPT, RPT)],
            out_hbm.at[pl.ds(c * NPAD + s * RPT, RPT)],
        )

    return deg_kernel


def _make_msg_kernel(E):
    ept = E // NUM_TILES
    assert ept % CHUNK == 0
    n_chunks = ept // CHUNK

    @functools.partial(
        pl.kernel,
        out_type=jax.ShapeDtypeStruct((NUM_CORES * NPAD, F), jnp.float32),
        mesh=_sc_mesh(),
        scratch_types=[
            pltpu.VMEM((CHUNK,), jnp.int32),
            pltpu.VMEM((CHUNK,), jnp.int32),
            pltpu.VMEM((CHUNK, F), jnp.float32),
            pltpu.VMEM_SHARED((NPAD, F), jnp.float32),
            pltpu.SemaphoreType.DMA,
        ],
    )
    def msg_kernel(hs_hbm, src_hbm, dst_hbm, zeros_hbm, out_hbm,
                   sidx, didx, rows, acc, sem):
        c = lax.axis_index("c")
        s = lax.axis_index("s")
        tile = c * NUM_SUBCORES + s
        pltpu.sync_copy(
            zeros_hbm.at[pl.ds(s * RPT, RPT)], acc.at[pl.ds(s * RPT, RPT)]
        )
        plsc.subcore_barrier()
        base = tile * ept

        def body(i, carry):
            off = base + i * CHUNK
            pltpu.sync_copy(src_hbm.at[pl.ds(off, CHUNK)], sidx)
            pltpu.sync_copy(dst_hbm.at[pl.ds(off, CHUNK)], didx)
            pltpu.async_copy(hs_hbm.at[sidx], rows, sem).wait()
            pltpu.sync_copy(rows, acc.at[didx], add=True)
            return carry

        lax.fori_loop(0, n_chunks, body, 0)
        plsc.subcore_barrier()
        pltpu.sync_copy(
            acc.at[pl.ds(s * RPT, RPT)],
            out_hbm.at[pl.ds(c * NPAD + s * RPT, RPT)],
        )

    return msg_kernel


def _dinv(degA_ref, degB_ref):
    deg = degA_ref[:, 0:1] + degB_ref[:, 0:1] + 1.0
    return lax.rsqrt(deg)


def _tc_input(degA, degB, x, mask2, W0):
    def body(degA_ref, degB_ref, x_ref, mask_ref, w_ref, hs_ref):
        dinv = _dinv(degA_ref, degB_ref)
        h = x_ref[...] * mask_ref[...]
        hs_ref[...] = jnp.dot(
            h, w_ref[...], preferred_element_type=jnp.float32) * dinv

    return pl.pallas_call(
        body, out_shape=jax.ShapeDtypeStruct((N, F), jnp.float32)
    )(degA, degB, x, mask2, W0)


def _bn_relu_mask(u_ref_vals, g, be, mask):
    mu = jnp.mean(u_ref_vals, axis=0, keepdims=True)
    var = jnp.mean((u_ref_vals - mu) ** 2, axis=0, keepdims=True)
    h = (u_ref_vals - mu) * lax.rsqrt(var + 1e-5) * g + be
    return jnp.maximum(h, 0.0) * mask


def _tc_layer(degA, degB, aggA, aggB, hs, b2, g2, be2, mask2, Wn):
    def body(degA_ref, degB_ref, aggA_ref, aggB_ref, hs_ref, b_ref, g_ref,
             be_ref, mask_ref, w_ref, out_ref):
        dinv = _dinv(degA_ref, degB_ref)
        u = dinv * (aggA_ref[...] + aggB_ref[...] + hs_ref[...]) + b_ref[...]
        h = _bn_relu_mask(u, g_ref[...], be_ref[...], mask_ref[...])
        out_ref[...] = jnp.dot(
            h, w_ref[...], preferred_element_type=jnp.float32) * dinv

    return pl.pallas_call(
        body, out_shape=jax.ShapeDtypeStruct((N, F), jnp.float32)
    )(degA, degB, aggA, aggB, hs, b2, g2, be2, mask2, Wn)


def _tc_final(degA, degB, aggA, aggB, hs, b2, g2, be2, mask2, batch2,
              Wout, bout2):
    def body(degA_ref, degB_ref, aggA_ref, aggB_ref, hs_ref, b_ref, g_ref,
             be_ref, mask_ref, batch_ref, wout_ref, bout_ref, out_ref):
        dinv = _dinv(degA_ref, degB_ref)
        u = dinv * (aggA_ref[...] + aggB_ref[...] + hs_ref[...]) + b_ref[...]
        h = _bn_relu_mask(u, g_ref[...], be_ref[...], mask_ref[...])
        onehot = (batch_ref[...] == lax.broadcasted_iota(
            jnp.int32, (N, G), 1)).astype(jnp.float32)
        dn = (((0,), (0,)), ((), ()))
        pooled = lax.dot_general(
            onehot, h, dn, preferred_element_type=jnp.float32)
        counts = lax.dot_general(
            onehot, jnp.ones((N, 1), jnp.float32), dn,
            preferred_element_type=jnp.float32)
        out_ref[...] = (
            jnp.dot(pooled, wout_ref[...], preferred_element_type=jnp.float32)
            / jnp.maximum(counts, 1.0)
            + bout_ref[...]
        )

    return pl.pallas_call(
        body, out_shape=jax.ShapeDtypeStruct((G, 1), jnp.float32)
    )(degA, degB, aggA, aggB, hs, b2, g2, be2, mask2, batch2, Wout, bout2)


def kernel(x, edge_index, mask, batch, W0, b0, g0, be0, W1, b1, g1, be1,
           W2, b2, g2, be2, Wout, bout):
    E = edge_index.shape[1]
    src = edge_index[0]
    dst = edge_index[1]
    mask2 = mask[:, None]
    batch2 = batch[:, None]
    zeros_deg = jnp.zeros((NPAD, DEG_W), jnp.float32)
    ones_chunk = jnp.ones((CHUNK, DEG_W), jnp.float32)
    zeros_big = jnp.zeros((NPAD, F), jnp.float32)

    deg_parts = _make_deg_kernel(E)(dst, ones_chunk, zeros_deg)
    degA, degB = deg_parts[:N], deg_parts[NPAD:NPAD + N]

    msg = _make_msg_kernel(E)

    hs = _tc_input(degA, degB, x, mask2, W0)
    agg = msg(hs, src, dst, zeros_big)
    hs = _tc_layer(degA, degB, agg[:N], agg[NPAD:NPAD + N], hs,
                   b0[None, :], g0[None, :], be0[None, :], mask2, W1)
    agg = msg(hs, src, dst, zeros_big)
    hs = _tc_layer(degA, degB, agg[:N], agg[NPAD:NPAD + N], hs,
                   b1[None, :], g1[None, :], be1[None, :], mask2, W2)
    agg = msg(hs, src, dst, zeros_big)
    out = _tc_final(degA, degB, agg[:N], agg[NPAD:NPAD + N], hs,
                    b2[None, :], g2[None, :], be2[None, :], mask2, batch2,
                    Wout, bout[None, :])
    return out.reshape(G)
